# baseline (speedup 1.0000x reference)
import functools

import jax
import jax.numpy as jnp
from jax import lax
from jax.experimental import pallas as pl
from jax.experimental.pallas import tpu as pltpu
from jax.experimental.pallas import tpu_sc as plsc

N_NODES = 10000
D = 128
NC = 2
NS = 16
L = 16
CHUNK = 128
NB = 2
SB = NB * CHUNK
ACC_ROWS = 10112
CNT_ROWS = 80
ROWS_PER_SUB = ACC_ROWS // NS
WB_OFFS = (0, 128, 256, 384, ROWS_PER_SUB - CHUNK)


def _sc_aggregate(x, src3, dst3, n_sb):
    mesh = plsc.VectorSubcoreMesh(core_axis_name="c", subcore_axis_name="s")

    @functools.partial(
        pl.kernel,
        out_type=(
            jax.ShapeDtypeStruct((NC, ACC_ROWS, D), jnp.float32),
            jax.ShapeDtypeStruct((NC, CNT_ROWS, D), jnp.float32),
        ),
        mesh=mesh,
        compiler_params=pltpu.CompilerParams(use_tc_tiling_on_sc=False,
                                             needs_layout_passes=False),
        scratch_types=[
            pltpu.VMEM((2, NB, CHUNK), jnp.int32),
            pltpu.VMEM((2, NB, CHUNK), jnp.int32),
            pltpu.VMEM((NB, CHUNK, D), jnp.float32),
            pltpu.VMEM((CNT_ROWS, D), jnp.float32),
            pltpu.VMEM((CNT_ROWS,), jnp.int32),
            pltpu.VMEM_SHARED((ACC_ROWS, D), jnp.float32),
            pltpu.VMEM_SHARED((CNT_ROWS, D), jnp.float32),
            pltpu.SemaphoreType.DMA((NB,)),
            pltpu.SemaphoreType.DMA((NB,)),
            pltpu.SemaphoreType.DMA((2,)),
        ],
    )
    def k(x_hbm, src_hbm, dst_hbm, part_hbm, cnt_hbm,
          sidx_v, didx_v, rows_v, hist_v, iota_v, acc_sh, cntacc_sh,
          gsem, ssem, isem):
        cid = lax.axis_index("c")
        sid = lax.axis_index("s")
        sub_row0 = sid * ROWS_PER_SUB
        sb0 = (cid * NS + sid) * n_sb

        def fill(i, _):
            for c in range(D // L):
                rows_v[0, i, pl.ds(c * L, L)] = jnp.zeros((L,), jnp.float32)
            return 0

        lax.fori_loop(0, CHUNK, fill, 0)

        def fill2(i, _):
            for c in range(D // L):
                hist_v[i, pl.ds(c * L, L)] = jnp.zeros((L,), jnp.float32)
            return 0

        lax.fori_loop(0, CNT_ROWS, fill2, 0)
        for c in range(CNT_ROWS // L):
            iota_v[pl.ds(c * L, L)] = lax.iota(jnp.int32, L) + (c * L)

        for woff in WB_OFFS:
            pltpu.sync_copy(rows_v.at[0], acc_sh.at[pl.ds(sub_row0 + woff,
                                                          CHUNK)])
        rpc = CNT_ROWS // NS
        pltpu.sync_copy(rows_v.at[0, pl.ds(0, rpc)],
                        cntacc_sh.at[pl.ds(sid * rpc, rpc)])
        plsc.subcore_barrier()

        def prefetch(b, slot):
            sb = sb0 + b
            pltpu.async_copy(src_hbm.at[sb], sidx_v.at[slot], isem.at[slot])
            pltpu.async_copy(dst_hbm.at[sb], didx_v.at[slot], isem.at[slot])

        def wait_idx(slot):
            pltpu.make_async_copy(src_hbm.at[0], sidx_v.at[slot],
                                  isem.at[slot]).wait()
            pltpu.make_async_copy(dst_hbm.at[0], didx_v.at[slot],
                                  isem.at[slot]).wait()

        def drain_rows(slot):
            for j in range(NB):
                pltpu.make_async_copy(rows_v.at[j],
                                      acc_sh.at[didx_v.at[slot, j]],
                                      ssem.at[j]).wait()

        prefetch(0, 0)

        def step(b, _):
            slot = lax.rem(b, 2)
            @pl.when(b > 0)
            def _():
                drain_rows(1 - slot)
            wait_idx(slot)

            @pl.when(b + 1 < n_sb)
            def _():
                prefetch(b + 1, 1 - slot)

            for j in range(NB):
                pltpu.async_copy(x_hbm.at[sidx_v.at[slot, j]], rows_v.at[j],
                                 gsem.at[j])
            for j in range(NB):
                for g in range(CHUNK // L):
                    d = didx_v[slot, j, pl.ds(g * L, L)]
                    cnts, last = plsc.scan_count(d)
                    plsc.addupdate_scatter(
                        hist_v,
                        [lax.shift_right_logical(d, 7),
                         lax.bitwise_and(d, 127)],
                        cnts.astype(jnp.float32), mask=last)
            for j in range(NB):
                pltpu.make_async_copy(x_hbm.at[sidx_v.at[slot, j]],
                                      rows_v.at[j], gsem.at[j]).wait()
                pltpu.async_copy(rows_v.at[j], acc_sh.at[didx_v.at[slot, j]],
                                 ssem.at[j], add=True)
            return 0

        lax.fori_loop(0, n_sb, step, 0)
        drain_rows((n_sb - 1) % 2)
        pltpu.sync_copy(hist_v, cntacc_sh.at[iota_v], add=True)
        plsc.subcore_barrier()

        for woff in WB_OFFS:
            r0 = sub_row0 + woff
            pltpu.sync_copy(acc_sh.at[pl.ds(r0, CHUNK)], rows_v.at[0])
            pltpu.sync_copy(rows_v.at[0], part_hbm.at[cid, pl.ds(r0, CHUNK)])

        @pl.when(sid == 0)
        def _():
            pltpu.sync_copy(cntacc_sh, hist_v)
            pltpu.sync_copy(hist_v, cnt_hbm.at[cid])

    return k(x, src3, dst3)


def _tc_finish_body(p_ref, c_ref, x_ref, w_ref, o_ref):
    cnt = c_ref[0] + c_ref[1]
    inv_col = (1.0 / jnp.maximum(cnt, 1.0)).reshape(D, 1)
    y = (p_ref[0] + p_ref[1]) * inv_col + x_ref[...]
    o_ref[...] = lax.dot_general(
        y, w_ref[...], (((1,), (1,)), ((), ())),
        preferred_element_type=jnp.float32)


def _tc_finish(part, cnt, xp, w_r):
    grid = (ACC_ROWS // CHUNK,)
    return pl.pallas_call(
        _tc_finish_body,
        grid=grid,
        in_specs=[
            pl.BlockSpec((NC, CHUNK, D), lambda i: (0, i, 0)),
            pl.BlockSpec((NC, D), lambda i: (0, i)),
            pl.BlockSpec((CHUNK, D), lambda i: (i, 0)),
            pl.BlockSpec((D, D), lambda i: (0, 0)),
        ],
        out_specs=pl.BlockSpec((CHUNK, D), lambda i: (i, 0)),
        out_shape=jax.ShapeDtypeStruct((ACC_ROWS, D), jnp.float32),
    )(part, cnt, xp, w_r)


@jax.jit
def kernel(x, edge_index, W_r):
    e = edge_index.shape[1]
    n_sb = -(-e // (NC * NS * SB))
    e_pad = NC * NS * n_sb * SB
    dst = edge_index[0].astype(jnp.int32)
    src = edge_index[1].astype(jnp.int32)
    pad = e_pad - e
    if pad:
        src = jnp.concatenate([src, jnp.zeros((pad,), jnp.int32)])
        dst = jnp.concatenate([dst, jnp.full((pad,), N_NODES, jnp.int32)])
    src3 = src.reshape(NC * NS * n_sb, NB, CHUNK)
    dst3 = dst.reshape(NC * NS * n_sb, NB, CHUNK)
    part, cnt = _sc_aggregate(x, src3, dst3, n_sb)
    cnt = cnt.reshape(NC, CNT_ROWS * D)
    xp = jnp.concatenate(
        [x, jnp.zeros((ACC_ROWS - N_NODES, D), jnp.float32)])
    return _tc_finish(part, cnt, xp, W_r)[:N_NODES]


# speedup vs baseline: 1.4646x; 1.4646x over previous
import functools

import jax
import jax.numpy as jnp
from jax import lax
from jax.experimental import pallas as pl
from jax.experimental.pallas import tpu as pltpu
from jax.experimental.pallas import tpu_sc as plsc

N_NODES = 10000
D = 128
DH = D // 2
NC = 2
NS = 16
L = 16
CHUNK = 128
NB = 8
SB = NB * CHUNK
ACC_ROWS = 10112
CNT_ROWS = 80
ROWS_PER_SUB = ACC_ROWS // NS
WB_OFFS = (0, 128, 256, 384, ROWS_PER_SUB - CHUNK)


def _sc_aggregate(xh, src2, dst2, n_sb):
    mesh = plsc.VectorSubcoreMesh(core_axis_name="c", subcore_axis_name="s")

    @functools.partial(
        pl.kernel,
        out_type=(
            jax.ShapeDtypeStruct((NC, ACC_ROWS, DH), jnp.float32),
            jax.ShapeDtypeStruct((NC, CNT_ROWS, D), jnp.float32),
        ),
        mesh=mesh,
        compiler_params=pltpu.CompilerParams(use_tc_tiling_on_sc=False,
                                             needs_layout_passes=False),
        scratch_types=[
            pltpu.VMEM((2, NB, CHUNK), jnp.int32),
            pltpu.VMEM((2, NB, CHUNK), jnp.int32),
            pltpu.VMEM((NB, CHUNK, DH), jnp.float32),
            pltpu.VMEM((CNT_ROWS, D), jnp.float32),
            pltpu.VMEM((CNT_ROWS,), jnp.int32),
            pltpu.VMEM_SHARED((ACC_ROWS, DH), jnp.float32),
            pltpu.VMEM_SHARED((CNT_ROWS, D), jnp.float32),
            pltpu.SemaphoreType.DMA((NB,)),
            pltpu.SemaphoreType.DMA((NB,)),
            pltpu.SemaphoreType.DMA((2,)),
        ],
    )
    def k(xh_hbm, src_hbm, dst_hbm, part_hbm, cnt_hbm,
          sidx_v, didx_v, rows_v, hist_v, iota_v, acc_sh, cntacc_sh,
          gsem, ssem, isem):
        cid = lax.axis_index("c")
        sid = lax.axis_index("s")
        sub_row0 = sid * ROWS_PER_SUB
        half = (n_sb + 1) // 2

        def fill(i, _):
            for c in range(DH // L):
                rows_v[0, i, pl.ds(c * L, L)] = jnp.zeros((L,), jnp.float32)
            return 0

        lax.fori_loop(0, CHUNK, fill, 0)

        def fill2(i, _):
            for c in range(D // L):
                hist_v[i, pl.ds(c * L, L)] = jnp.zeros((L,), jnp.float32)
            return 0

        lax.fori_loop(0, CNT_ROWS, fill2, 0)
        for c in range(CNT_ROWS // L):
            iota_v[pl.ds(c * L, L)] = lax.iota(jnp.int32, L) + (c * L)

        for woff in WB_OFFS:
            pltpu.sync_copy(rows_v.at[0],
                            acc_sh.at[pl.ds(sub_row0 + woff, CHUNK)])
        rpc = CNT_ROWS // NS
        pltpu.sync_copy(hist_v.at[pl.ds(0, rpc)],
                        cntacc_sh.at[pl.ds(sid * rpc, rpc)])
        plsc.subcore_barrier()

        def prefetch(b, slot):
            sb = sid * n_sb + b
            pltpu.async_copy(src_hbm.at[cid, sb], sidx_v.at[slot],
                             isem.at[slot])
            pltpu.async_copy(dst_hbm.at[sb], didx_v.at[slot], isem.at[slot])

        def wait_idx(slot):
            pltpu.make_async_copy(src_hbm.at[cid, 0], sidx_v.at[slot],
                                  isem.at[slot]).wait()
            pltpu.make_async_copy(dst_hbm.at[0], didx_v.at[slot],
                                  isem.at[slot]).wait()

        def drain_rows(slot):
            for j in range(NB):
                pltpu.make_async_copy(rows_v.at[j],
                                      acc_sh.at[didx_v.at[slot, j]],
                                      ssem.at[j]).wait()

        def counts_at(b):
            return lax.select(cid == 0, b < half, b >= half)

        prefetch(0, 0)

        def step(b, _):
            slot = lax.rem(b, 2)
            @pl.when(b > 0)
            def _():
                drain_rows(1 - slot)
            wait_idx(slot)

            @pl.when(b + 1 < n_sb)
            def _():
                prefetch(b + 1, 1 - slot)

            for j in range(NB):
                pltpu.async_copy(xh_hbm.at[sidx_v.at[slot, j]], rows_v.at[j],
                                 gsem.at[j])

            @pl.when(counts_at(b))
            def _():
                for j in range(NB):
                    for g in range(CHUNK // L):
                        d = didx_v[slot, j, pl.ds(g * L, L)]
                        cnts, last = plsc.scan_count(d)
                        plsc.addupdate_scatter(
                            hist_v,
                            [lax.shift_right_logical(d, 7),
                             lax.bitwise_and(d, 127)],
                            cnts.astype(jnp.float32), mask=last)

            for j in range(NB):
                pltpu.make_async_copy(xh_hbm.at[sidx_v.at[slot, j]],
                                      rows_v.at[j], gsem.at[j]).wait()
                pltpu.async_copy(rows_v.at[j], acc_sh.at[didx_v.at[slot, j]],
                                 ssem.at[j], add=True)
            return 0

        lax.fori_loop(0, n_sb, step, 0)
        drain_rows((n_sb - 1) % 2)
        pltpu.sync_copy(hist_v, cntacc_sh.at[iota_v], add=True)
        plsc.subcore_barrier()

        for woff in WB_OFFS:
            r0 = sub_row0 + woff
            pltpu.sync_copy(acc_sh.at[pl.ds(r0, CHUNK)], rows_v.at[0])
            pltpu.sync_copy(rows_v.at[0], part_hbm.at[cid, pl.ds(r0, CHUNK)])

        @pl.when(sid == 0)
        def _():
            pltpu.sync_copy(cntacc_sh, hist_v)
            pltpu.sync_copy(hist_v, cnt_hbm.at[cid])

    return k(xh, src2, dst2)


def _tc_finish_body(p_ref, c_ref, x_ref, w_ref, o_ref):
    cnt = c_ref[0] + c_ref[1]
    inv_col = (1.0 / jnp.maximum(cnt, 1.0)).reshape(D, 1)
    y_lo = p_ref[0] * inv_col + x_ref[:, :DH]
    y_hi = p_ref[1] * inv_col + x_ref[:, DH:]
    o_ref[...] = lax.dot_general(
        y_lo, w_ref[:, :DH], (((1,), (1,)), ((), ())),
        preferred_element_type=jnp.float32) + lax.dot_general(
        y_hi, w_ref[:, DH:], (((1,), (1,)), ((), ())),
        preferred_element_type=jnp.float32)


def _tc_finish(part, cnt, xp, w_r):
    grid = (ACC_ROWS // CHUNK,)
    return pl.pallas_call(
        _tc_finish_body,
        grid=grid,
        in_specs=[
            pl.BlockSpec((NC, CHUNK, DH), lambda i: (0, i, 0)),
            pl.BlockSpec((NC, D), lambda i: (0, i)),
            pl.BlockSpec((CHUNK, D), lambda i: (i, 0)),
            pl.BlockSpec((D, D), lambda i: (0, 0)),
        ],
        out_specs=pl.BlockSpec((CHUNK, D), lambda i: (i, 0)),
        out_shape=jax.ShapeDtypeStruct((ACC_ROWS, D), jnp.float32),
    )(part, cnt, xp, w_r)


@jax.jit
def kernel(x, edge_index, W_r):
    e = edge_index.shape[1]
    n_sb = -(-e // (NS * SB))
    e_pad = NS * n_sb * SB
    dst = edge_index[0].astype(jnp.int32)
    src = edge_index[1].astype(jnp.int32)
    pad = e_pad - e
    if pad:
        src = jnp.concatenate([src, jnp.zeros((pad,), jnp.int32)])
        dst = jnp.concatenate([dst, jnp.full((pad,), N_NODES, jnp.int32)])
    src2 = jnp.stack([src, src + N_NODES]).reshape(NC, NS * n_sb, NB, CHUNK)
    dst2 = dst.reshape(NS * n_sb, NB, CHUNK)
    xh = jnp.concatenate([x[:, :DH], x[:, DH:]], axis=0)
    part, cnt = _sc_aggregate(xh, src2, dst2, n_sb)
    cnt = cnt.reshape(NC, CNT_ROWS * D)
    xp = jnp.concatenate(
        [x, jnp.zeros((ACC_ROWS - N_NODES, D), jnp.float32)])
    return _tc_finish(part, cnt, xp, W_r)[:N_NODES]


# speedup vs baseline: 2.3429x; 1.5997x over previous
import functools

import jax
import jax.numpy as jnp
from jax import lax
from jax.experimental import pallas as pl
from jax.experimental.pallas import tpu as pltpu
from jax.experimental.pallas import tpu_sc as plsc

N_NODES = 10000
D = 128
DH = D // 2
NC = 2
NS = 16
L = 16
CHUNK = 128
NB = 4
SB = NB * CHUNK
ACC_ROWS = 10112
CNT_ROWS = 80
ROWS_PER_SUB = ACC_ROWS // NS
WB_OFFS = (0, 128, 256, 384, ROWS_PER_SUB - CHUNK)


def _sc_aggregate(xh, src2, dst2, n_sb):
    mesh = plsc.VectorSubcoreMesh(core_axis_name="c", subcore_axis_name="s")

    @functools.partial(
        pl.kernel,
        out_type=(
            jax.ShapeDtypeStruct((NC, ACC_ROWS, DH), jnp.float32),
            jax.ShapeDtypeStruct((NC, CNT_ROWS, D), jnp.float32),
        ),
        mesh=mesh,
        compiler_params=pltpu.CompilerParams(use_tc_tiling_on_sc=False,
                                             needs_layout_passes=False),
        scratch_types=[
            pltpu.VMEM((2, NB, CHUNK), jnp.int32),
            pltpu.VMEM((2, NB, CHUNK), jnp.int32),
            pltpu.VMEM((NB, CHUNK, DH), jnp.float32),
            pltpu.VMEM((CNT_ROWS, D), jnp.float32),
            pltpu.VMEM((CNT_ROWS,), jnp.int32),
            pltpu.VMEM_SHARED((N_NODES, DH), jnp.float32),
            pltpu.VMEM_SHARED((ACC_ROWS, DH), jnp.float32),
            pltpu.VMEM_SHARED((CNT_ROWS, D), jnp.float32),
            pltpu.SemaphoreType.DMA((NB,)),
            pltpu.SemaphoreType.DMA((NB,)),
            pltpu.SemaphoreType.DMA((2,)),
        ],
    )
    def k(xh_hbm, src_hbm, dst_hbm, part_hbm, cnt_hbm,
          sidx_v, didx_v, rows_v, hist_v, iota_v, xsp_sh, acc_sh, cntacc_sh,
          gsem, ssem, isem):
        cid = lax.axis_index("c")
        sid = lax.axis_index("s")
        sub_row0 = sid * ROWS_PER_SUB
        half = (n_sb + 1) // 2

        def fill(i, _):
            for c in range(DH // L):
                rows_v[0, i, pl.ds(c * L, L)] = jnp.zeros((L,), jnp.float32)
            return 0

        lax.fori_loop(0, CHUNK, fill, 0)

        def fill2(i, _):
            for c in range(D // L):
                hist_v[i, pl.ds(c * L, L)] = jnp.zeros((L,), jnp.float32)
            return 0

        lax.fori_loop(0, CNT_ROWS, fill2, 0)
        for c in range(CNT_ROWS // L):
            iota_v[pl.ds(c * L, L)] = lax.iota(jnp.int32, L) + (c * L)

        for woff in WB_OFFS:
            pltpu.sync_copy(rows_v.at[0],
                            acc_sh.at[pl.ds(sub_row0 + woff, CHUNK)])
        rpc = CNT_ROWS // NS
        pltpu.sync_copy(hist_v.at[pl.ds(0, rpc)],
                        cntacc_sh.at[pl.ds(sid * rpc, rpc)])
        xrows = N_NODES // NS
        pltpu.sync_copy(xh_hbm.at[pl.ds(cid * N_NODES + sid * xrows, xrows)],
                        xsp_sh.at[pl.ds(sid * xrows, xrows)])
        plsc.subcore_barrier()

        def prefetch(b, slot):
            sb = sid * n_sb + b
            pltpu.async_copy(src_hbm.at[sb], sidx_v.at[slot], isem.at[slot])
            pltpu.async_copy(dst_hbm.at[sb], didx_v.at[slot], isem.at[slot])

        def wait_idx(slot):
            pltpu.make_async_copy(src_hbm.at[0], sidx_v.at[slot],
                                  isem.at[slot]).wait()
            pltpu.make_async_copy(dst_hbm.at[0], didx_v.at[slot],
                                  isem.at[slot]).wait()

        def drain_rows(slot):
            for j in range(NB):
                pltpu.make_async_copy(rows_v.at[j],
                                      acc_sh.at[didx_v.at[slot, j]],
                                      ssem.at[j]).wait()

        def counts_at(b):
            return lax.select(cid == 0, b < half, b >= half)

        prefetch(0, 0)

        def step(b, _):
            slot = lax.rem(b, 2)
            @pl.when(b > 0)
            def _():
                drain_rows(1 - slot)
            wait_idx(slot)

            @pl.when(b + 1 < n_sb)
            def _():
                prefetch(b + 1, 1 - slot)

            for j in range(NB):
                pltpu.async_copy(xsp_sh.at[sidx_v.at[slot, j]], rows_v.at[j],
                                 gsem.at[j])
            for j in range(NB):
                pltpu.make_async_copy(xsp_sh.at[sidx_v.at[slot, j]],
                                      rows_v.at[j], gsem.at[j]).wait()
                pltpu.async_copy(rows_v.at[j], acc_sh.at[didx_v.at[slot, j]],
                                 ssem.at[j], add=True)

            @pl.when(counts_at(b))
            def _():
                for j in range(NB):
                    for g in range(CHUNK // L):
                        d = didx_v[slot, j, pl.ds(g * L, L)]
                        cnts, last = plsc.scan_count(d)
                        plsc.addupdate_scatter(
                            hist_v,
                            [lax.shift_right_logical(d, 7),
                             lax.bitwise_and(d, 127)],
                            cnts.astype(jnp.float32), mask=last)
            return 0

        lax.fori_loop(0, n_sb, step, 0)
        drain_rows((n_sb - 1) % 2)
        pltpu.sync_copy(hist_v, cntacc_sh.at[iota_v], add=True)
        plsc.subcore_barrier()

        for woff in WB_OFFS:
            r0 = sub_row0 + woff
            pltpu.sync_copy(acc_sh.at[pl.ds(r0, CHUNK)], rows_v.at[0])
            pltpu.sync_copy(rows_v.at[0], part_hbm.at[cid, pl.ds(r0, CHUNK)])

        @pl.when(sid == 0)
        def _():
            pltpu.sync_copy(cntacc_sh, hist_v)
            pltpu.sync_copy(hist_v, cnt_hbm.at[cid])

    return k(xh, src2, dst2)


def _tc_finish_body(p_ref, c_ref, x_ref, w_ref, o_ref):
    cnt = c_ref[0] + c_ref[1]
    inv_col = (1.0 / jnp.maximum(cnt, 1.0)).reshape(D, 1)
    y_lo = p_ref[0] * inv_col + x_ref[:, :DH]
    y_hi = p_ref[1] * inv_col + x_ref[:, DH:]
    o_ref[...] = lax.dot_general(
        y_lo, w_ref[:, :DH], (((1,), (1,)), ((), ())),
        preferred_element_type=jnp.float32) + lax.dot_general(
        y_hi, w_ref[:, DH:], (((1,), (1,)), ((), ())),
        preferred_element_type=jnp.float32)


def _tc_finish(part, cnt, xp, w_r):
    grid = (ACC_ROWS // CHUNK,)
    return pl.pallas_call(
        _tc_finish_body,
        grid=grid,
        in_specs=[
            pl.BlockSpec((NC, CHUNK, DH), lambda i: (0, i, 0)),
            pl.BlockSpec((NC, D), lambda i: (0, i)),
            pl.BlockSpec((CHUNK, D), lambda i: (i, 0)),
            pl.BlockSpec((D, D), lambda i: (0, 0)),
        ],
        out_specs=pl.BlockSpec((CHUNK, D), lambda i: (i, 0)),
        out_shape=jax.ShapeDtypeStruct((ACC_ROWS, D), jnp.float32),
    )(part, cnt, xp, w_r)


@jax.jit
def kernel(x, edge_index, W_r):
    e = edge_index.shape[1]
    n_sb = -(-e // (NS * SB))
    e_pad = NS * n_sb * SB
    dst = edge_index[0].astype(jnp.int32)
    src = edge_index[1].astype(jnp.int32)
    pad = e_pad - e
    if pad:
        src = jnp.concatenate([src, jnp.zeros((pad,), jnp.int32)])
        dst = jnp.concatenate([dst, jnp.full((pad,), N_NODES, jnp.int32)])
    src2 = src.reshape(NS * n_sb, NB, CHUNK)
    dst2 = dst.reshape(NS * n_sb, NB, CHUNK)
    xh = jnp.concatenate([x[:, :DH], x[:, DH:]], axis=0)
    part, cnt = _sc_aggregate(xh, src2, dst2, n_sb)
    cnt = cnt.reshape(NC, CNT_ROWS * D)
    xp = jnp.concatenate(
        [x, jnp.zeros((ACC_ROWS - N_NODES, D), jnp.float32)])
    return _tc_finish(part, cnt, xp, W_r)[:N_NODES]


# speedup vs baseline: 2.5801x; 1.1013x over previous
import functools

import jax
import jax.numpy as jnp
from jax import lax
from jax.experimental import pallas as pl
from jax.experimental.pallas import tpu as pltpu
from jax.experimental.pallas import tpu_sc as plsc

N_NODES = 10000
D = 128
DH = D // 2
NC = 2
NS = 16
L = 16
CHUNK = 128
NB = 4
SB = NB * CHUNK
ACC_ROWS = 10112
CNT_ROWS = 80
ROWS_PER_SUB = ACC_ROWS // NS
WB_OFFS = (0, 128, 256, 384, ROWS_PER_SUB - CHUNK)


def _sc_aggregate(x, src2, dst2, n_sb):
    mesh = plsc.VectorSubcoreMesh(core_axis_name="c", subcore_axis_name="s")

    @functools.partial(
        pl.kernel,
        out_type=(
            jax.ShapeDtypeStruct((NC, ACC_ROWS, DH), jnp.float32),
            jax.ShapeDtypeStruct((NC, CNT_ROWS, D), jnp.float32),
        ),
        mesh=mesh,
        compiler_params=pltpu.CompilerParams(use_tc_tiling_on_sc=False,
                                             needs_layout_passes=False),
        scratch_types=[
            pltpu.VMEM((2, NB, CHUNK), jnp.int32),
            pltpu.VMEM((2, NB, CHUNK), jnp.int32),
            pltpu.VMEM((NB, CHUNK, DH), jnp.float32),
            pltpu.VMEM((CNT_ROWS, D), jnp.float32),
            pltpu.VMEM((CNT_ROWS,), jnp.int32),
            pltpu.VMEM_SHARED((N_NODES, DH), jnp.float32),
            pltpu.VMEM_SHARED((ACC_ROWS, DH), jnp.float32),
            pltpu.VMEM_SHARED((CNT_ROWS, D), jnp.float32),
            pltpu.SemaphoreType.DMA((NB,)),
            pltpu.SemaphoreType.DMA((NB,)),
            pltpu.SemaphoreType.DMA((2,)),
        ],
    )
    def k(x_hbm, src_hbm, dst_hbm, part_hbm, cnt_hbm,
          sidx_v, didx_v, rows_v, hist_v, iota_v, xsp_sh, acc_sh, cntacc_sh,
          gsem, ssem, isem):
        cid = lax.axis_index("c")
        sid = lax.axis_index("s")
        sub_row0 = sid * ROWS_PER_SUB
        half = (n_sb + 1) // 2

        def fill(i, _):
            for c in range(DH // L):
                rows_v[0, i, pl.ds(c * L, L)] = jnp.zeros((L,), jnp.float32)
            return 0

        lax.fori_loop(0, CHUNK, fill, 0)

        def fill2(i, _):
            for c in range(D // L):
                hist_v[i, pl.ds(c * L, L)] = jnp.zeros((L,), jnp.float32)
            return 0

        lax.fori_loop(0, CNT_ROWS, fill2, 0)
        for c in range(CNT_ROWS // L):
            iota_v[pl.ds(c * L, L)] = lax.iota(jnp.int32, L) + (c * L)

        for woff in WB_OFFS:
            pltpu.sync_copy(rows_v.at[0],
                            acc_sh.at[pl.ds(sub_row0 + woff, CHUNK)])
        rpc = CNT_ROWS // NS
        pltpu.sync_copy(hist_v.at[pl.ds(0, rpc)],
                        cntacc_sh.at[pl.ds(sid * rpc, rpc)])
        xrows = N_NODES // NS
        pltpu.sync_copy(x_hbm.at[pl.ds(sid * xrows, xrows),
                                 pl.ds(cid * DH, DH)],
                        xsp_sh.at[pl.ds(sid * xrows, xrows)])
        plsc.subcore_barrier()

        def prefetch(b, slot):
            sb = sid * n_sb + b
            pltpu.async_copy(src_hbm.at[sb], sidx_v.at[slot], isem.at[slot])
            pltpu.async_copy(dst_hbm.at[sb], didx_v.at[slot], isem.at[slot])

        def wait_idx(slot):
            pltpu.make_async_copy(src_hbm.at[0], sidx_v.at[slot],
                                  isem.at[slot]).wait()
            pltpu.make_async_copy(dst_hbm.at[0], didx_v.at[slot],
                                  isem.at[slot]).wait()

        def drain_rows(slot):
            for j in range(NB):
                pltpu.make_async_copy(rows_v.at[j],
                                      acc_sh.at[didx_v.at[slot, j]],
                                      ssem.at[j]).wait()

        def counts_at(b):
            return lax.select(cid == 0, b < half, b >= half)

        prefetch(0, 0)

        def step(b, _):
            slot = lax.rem(b, 2)
            @pl.when(b > 0)
            def _():
                drain_rows(1 - slot)
            wait_idx(slot)

            @pl.when(b + 1 < n_sb)
            def _():
                prefetch(b + 1, 1 - slot)

            for j in range(NB):
                pltpu.async_copy(xsp_sh.at[sidx_v.at[slot, j]], rows_v.at[j],
                                 gsem.at[j])
            for j in range(NB):
                pltpu.make_async_copy(xsp_sh.at[sidx_v.at[slot, j]],
                                      rows_v.at[j], gsem.at[j]).wait()
                pltpu.async_copy(rows_v.at[j], acc_sh.at[didx_v.at[slot, j]],
                                 ssem.at[j], add=True)

            @pl.when(counts_at(b))
            def _():
                for j in range(NB):
                    for g in range(CHUNK // L):
                        d = didx_v[slot, j, pl.ds(g * L, L)]
                        cnts, last = plsc.scan_count(d)
                        plsc.addupdate_scatter(
                            hist_v,
                            [lax.shift_right_logical(d, 7),
                             lax.bitwise_and(d, 127)],
                            cnts.astype(jnp.float32), mask=last)
            return 0

        lax.fori_loop(0, n_sb, step, 0)
        drain_rows((n_sb - 1) % 2)
        pltpu.sync_copy(hist_v, cntacc_sh.at[iota_v], add=True)
        plsc.subcore_barrier()

        for woff in WB_OFFS:
            r0 = sub_row0 + woff
            pltpu.sync_copy(acc_sh.at[pl.ds(r0, CHUNK)], rows_v.at[0])
            pltpu.sync_copy(rows_v.at[0], part_hbm.at[cid, pl.ds(r0, CHUNK)])

        @pl.when(sid == 0)
        def _():
            pltpu.sync_copy(cntacc_sh, hist_v)
            pltpu.sync_copy(hist_v, cnt_hbm.at[cid])

    return k(x, src2, dst2)


def _tc_finish_body(p_ref, c_ref, x_ref, w_ref, o_ref):
    cnt = c_ref[0] + c_ref[1]
    inv_col = (1.0 / jnp.maximum(cnt, 1.0)).reshape(D, 1)
    y_lo = p_ref[0] * inv_col + x_ref[:, :DH]
    y_hi = p_ref[1] * inv_col + x_ref[:, DH:]
    o_ref[...] = lax.dot_general(
        y_lo, w_ref[:, :DH], (((1,), (1,)), ((), ())),
        preferred_element_type=jnp.float32) + lax.dot_general(
        y_hi, w_ref[:, DH:], (((1,), (1,)), ((), ())),
        preferred_element_type=jnp.float32)


def _tc_finish(part, cnt, x, w_r):
    grid = (ACC_ROWS // CHUNK,)
    return pl.pallas_call(
        _tc_finish_body,
        grid=grid,
        in_specs=[
            pl.BlockSpec((NC, CHUNK, DH), lambda i: (0, i, 0)),
            pl.BlockSpec((NC, D), lambda i: (0, i)),
            pl.BlockSpec((CHUNK, D), lambda i: (i, 0)),
            pl.BlockSpec((D, D), lambda i: (0, 0)),
        ],
        out_specs=pl.BlockSpec((CHUNK, D), lambda i: (i, 0)),
        out_shape=jax.ShapeDtypeStruct((N_NODES, D), jnp.float32),
    )(part, cnt, x, w_r)


@jax.jit
def kernel(x, edge_index, W_r):
    e = edge_index.shape[1]
    n_sb = -(-e // (NS * SB))
    e_pad = NS * n_sb * SB
    dst = edge_index[0].astype(jnp.int32)
    src = edge_index[1].astype(jnp.int32)
    pad = e_pad - e
    if pad:
        src = jnp.concatenate([src, jnp.zeros((pad,), jnp.int32)])
        dst = jnp.concatenate([dst, jnp.full((pad,), N_NODES, jnp.int32)])
    src2 = src.reshape(NS * n_sb, NB, CHUNK)
    dst2 = dst.reshape(NS * n_sb, NB, CHUNK)
    part, cnt = _sc_aggregate(x, src2, dst2, n_sb)
    cnt = cnt.reshape(NC, CNT_ROWS * D)
    return _tc_finish(part, cnt, x, W_r)


# speedup vs baseline: 2.5949x; 1.0057x over previous
import functools

import jax
import jax.numpy as jnp
from jax import lax
from jax.experimental import pallas as pl
from jax.experimental.pallas import tpu as pltpu
from jax.experimental.pallas import tpu_sc as plsc

N_NODES = 10000
D = 128
DH = D // 2
NC = 2
NS = 16
L = 16
CHUNK = 128
NB = 4
SB = NB * CHUNK
ACC_ROWS = 10112
CNT_ROWS = 80
ROWS_PER_SUB = ACC_ROWS // NS
WB_OFFS = (0, 128, 256, 384, ROWS_PER_SUB - CHUNK)


def _sc_aggregate(x, src2, dst2, n_sb):
    mesh = plsc.VectorSubcoreMesh(core_axis_name="c", subcore_axis_name="s")

    @functools.partial(
        pl.kernel,
        out_type=(
            jax.ShapeDtypeStruct((NC, ACC_ROWS, DH), jnp.float32),
            jax.ShapeDtypeStruct((NC, CNT_ROWS, D), jnp.float32),
        ),
        mesh=mesh,
        compiler_params=pltpu.CompilerParams(use_tc_tiling_on_sc=False,
                                             needs_layout_passes=False),
        scratch_types=[
            pltpu.VMEM((2, NB, CHUNK), jnp.int32),
            pltpu.VMEM((2, NB, CHUNK), jnp.int32),
            pltpu.VMEM((NB, CHUNK, DH), jnp.float32),
            pltpu.VMEM((CNT_ROWS, D), jnp.float32),
            pltpu.VMEM((CNT_ROWS,), jnp.int32),
            pltpu.VMEM_SHARED((N_NODES, DH), jnp.float32),
            pltpu.VMEM_SHARED((ACC_ROWS, DH), jnp.float32),
            pltpu.VMEM_SHARED((CNT_ROWS, D), jnp.float32),
            pltpu.SemaphoreType.DMA((NB,)),
            pltpu.SemaphoreType.DMA((NB,)),
            pltpu.SemaphoreType.DMA((2,)),
        ],
    )
    def k(x_hbm, src_hbm, dst_hbm, part_hbm, cnt_hbm,
          sidx_v, didx_v, rows_v, hist_v, iota_v, xsp_sh, acc_sh, cntacc_sh,
          gsem, ssem, isem):
        cid = lax.axis_index("c")
        sid = lax.axis_index("s")
        sub_row0 = sid * ROWS_PER_SUB
        half = (n_sb + 1) // 2

        def fill(i, _):
            for c in range(DH // L):
                rows_v[0, i, pl.ds(c * L, L)] = jnp.zeros((L,), jnp.float32)
            return 0

        lax.fori_loop(0, CHUNK, fill, 0)

        def fill2(i, _):
            for c in range(D // L):
                hist_v[i, pl.ds(c * L, L)] = jnp.zeros((L,), jnp.float32)
            return 0

        lax.fori_loop(0, CNT_ROWS, fill2, 0)
        for c in range(CNT_ROWS // L):
            iota_v[pl.ds(c * L, L)] = lax.iota(jnp.int32, L) + (c * L)

        for woff in WB_OFFS:
            pltpu.sync_copy(rows_v.at[0],
                            acc_sh.at[pl.ds(sub_row0 + woff, CHUNK)])
        rpc = CNT_ROWS // NS
        pltpu.sync_copy(hist_v.at[pl.ds(0, rpc)],
                        cntacc_sh.at[pl.ds(sid * rpc, rpc)])
        xrows = N_NODES // NS
        pltpu.sync_copy(x_hbm.at[pl.ds(sid * xrows, xrows),
                                 pl.ds(cid * DH, DH)],
                        xsp_sh.at[pl.ds(sid * xrows, xrows)])
        plsc.subcore_barrier()

        def prefetch(b, slot):
            sb = sid * n_sb + b
            pltpu.async_copy(src_hbm.at[sb], sidx_v.at[slot], isem.at[slot])
            pltpu.async_copy(dst_hbm.at[sb], didx_v.at[slot], isem.at[slot])

        def wait_idx(slot):
            pltpu.make_async_copy(src_hbm.at[0], sidx_v.at[slot],
                                  isem.at[slot]).wait()
            pltpu.make_async_copy(dst_hbm.at[0], didx_v.at[slot],
                                  isem.at[slot]).wait()

        def drain_rows(slot):
            for j in range(NB):
                pltpu.make_async_copy(rows_v.at[j],
                                      acc_sh.at[didx_v.at[slot, j]],
                                      ssem.at[j]).wait()

        def counts_at(b):
            return lax.select(cid == 0, b < half, b >= half)

        prefetch(0, 0)

        def step(b, _):
            slot = lax.rem(b, 2)
            @pl.when(b > 0)
            def _():
                drain_rows(1 - slot)
            wait_idx(slot)

            @pl.when(b + 1 < n_sb)
            def _():
                prefetch(b + 1, 1 - slot)

            for j in range(NB):
                pltpu.async_copy(xsp_sh.at[sidx_v.at[slot, j]], rows_v.at[j],
                                 gsem.at[j])
            for j in range(NB):
                pltpu.make_async_copy(xsp_sh.at[sidx_v.at[slot, j]],
                                      rows_v.at[j], gsem.at[j]).wait()
                pltpu.async_copy(rows_v.at[j], acc_sh.at[didx_v.at[slot, j]],
                                 ssem.at[j], add=True)

            @pl.when(counts_at(b))
            def _():
                for j in range(NB):
                    for g in range(CHUNK // L):
                        d = didx_v[slot, j, pl.ds(g * L, L)]
                        cnts, last = plsc.scan_count(d)
                        plsc.addupdate_scatter(
                            hist_v,
                            [lax.shift_right_logical(d, 7),
                             lax.bitwise_and(d, 127)],
                            cnts.astype(jnp.float32), mask=last)
            return 0

        lax.fori_loop(0, n_sb, step, 0)
        drain_rows((n_sb - 1) % 2)
        pltpu.sync_copy(hist_v, cntacc_sh.at[iota_v], add=True)
        plsc.subcore_barrier()

        for woff in WB_OFFS:
            r0 = sub_row0 + woff
            pltpu.sync_copy(acc_sh.at[pl.ds(r0, CHUNK)], rows_v.at[0])
            pltpu.sync_copy(rows_v.at[0], part_hbm.at[cid, pl.ds(r0, CHUNK)])

        @pl.when(sid == 0)
        def _():
            pltpu.sync_copy(cntacc_sh, hist_v)
            pltpu.sync_copy(hist_v, cnt_hbm.at[cid])

    return k(x, src2, dst2)


def _tc_xw_body(x_ref, w_ref, o_ref):
    o_ref[...] = lax.dot_general(
        x_ref[...], w_ref[...], (((1,), (1,)), ((), ())),
        preferred_element_type=jnp.float32)


def _tc_xw(x, w_r):
    blk = 1000
    return pl.pallas_call(
        _tc_xw_body,
        grid=(N_NODES // blk,),
        in_specs=[
            pl.BlockSpec((blk, D), lambda i: (i, 0)),
            pl.BlockSpec((D, D), lambda i: (0, 0)),
        ],
        out_specs=pl.BlockSpec((blk, D), lambda i: (i, 0)),
        out_shape=jax.ShapeDtypeStruct((N_NODES, D), jnp.float32),
    )(x, w_r)


def _tc_finish_body(p_ref, c_ref, xw_ref, w_ref, o_ref):
    cnt = c_ref[0] + c_ref[1]
    inv_col = (1.0 / jnp.maximum(cnt, 1.0)).reshape(D, 1)
    y_lo = p_ref[0] * inv_col
    y_hi = p_ref[1] * inv_col
    o_ref[...] = xw_ref[...] + lax.dot_general(
        y_lo, w_ref[:, :DH], (((1,), (1,)), ((), ())),
        preferred_element_type=jnp.float32) + lax.dot_general(
        y_hi, w_ref[:, DH:], (((1,), (1,)), ((), ())),
        preferred_element_type=jnp.float32)


def _tc_finish(part, cnt, x, w_r):
    grid = (ACC_ROWS // CHUNK,)
    return pl.pallas_call(
        _tc_finish_body,
        grid=grid,
        in_specs=[
            pl.BlockSpec((NC, CHUNK, DH), lambda i: (0, i, 0)),
            pl.BlockSpec((NC, D), lambda i: (0, i)),
            pl.BlockSpec((CHUNK, D), lambda i: (i, 0)),
            pl.BlockSpec((D, D), lambda i: (0, 0)),
        ],
        out_specs=pl.BlockSpec((CHUNK, D), lambda i: (i, 0)),
        out_shape=jax.ShapeDtypeStruct((N_NODES, D), jnp.float32),
    )(part, cnt, x, w_r)


@jax.jit
def kernel(x, edge_index, W_r):
    e = edge_index.shape[1]
    n_sb = -(-e // (NS * SB))
    e_pad = NS * n_sb * SB
    dst = edge_index[0].astype(jnp.int32)
    src = edge_index[1].astype(jnp.int32)
    pad = e_pad - e
    if pad:
        src = jnp.concatenate([src, jnp.zeros((pad,), jnp.int32)])
        dst = jnp.concatenate([dst, jnp.full((pad,), N_NODES, jnp.int32)])
    src2 = src.reshape(NS * n_sb, NB, CHUNK)
    dst2 = dst.reshape(NS * n_sb, NB, CHUNK)
    xw = _tc_xw(x, W_r)
    part, cnt = _sc_aggregate(x, src2, dst2, n_sb)
    cnt = cnt.reshape(NC, CNT_ROWS * D)
    return _tc_finish(part, cnt, xw, W_r)
